# baseline (speedup 1.0000x reference)
import functools

import jax
import jax.numpy as jnp
from jax import lax
from jax.experimental import pallas as pl
from jax.experimental.pallas import tpu as pltpu
from jax.experimental.pallas import tpu_sc as plsc

NC = 2
NS = 16
NW = NC * NS

CH = 512
G = CH // 16

TCOLS = 512


@functools.lru_cache(maxsize=None)
def _make_repack(vocab: int, dim: int):
    n_full = vocab // TCOLS
    rem = vocab - n_full * TCOLS
    per_w = (n_full + NW - 1) // NW
    if per_w % 2:
        per_w += 1
    last_s = n_full - 1
    pitch = TCOLS + 1

    mesh = plsc.VectorSubcoreMesh(core_axis_name="c", subcore_axis_name="s")

    @functools.partial(
        pl.kernel,
        out_type=jax.ShapeDtypeStruct((vocab * dim,), jnp.float32),
        mesh=mesh,
        scratch_types=[
            pltpu.VMEM((dim, TCOLS), jnp.float32),
            pltpu.VMEM((dim, TCOLS), jnp.float32),
            pltpu.VMEM((dim * pitch,), jnp.float32),
            pltpu.VMEM((TCOLS * dim,), jnp.float32),
            pltpu.VMEM((TCOLS * dim,), jnp.float32),
            pltpu.VMEM((dim, max(rem, 16)), jnp.float32),
            pltpu.SemaphoreType.DMA,
            pltpu.SemaphoreType.DMA,
            pltpu.SemaphoreType.DMA,
            pltpu.SemaphoreType.DMA,
        ],
        compiler_params=pltpu.CompilerParams(
            use_tc_tiling_on_sc=True, needs_layout_passes=False,
            disable_bounds_checks=True
        ),
    )
    def repack_kernel(tt_hbm, out_hbm, tb0, tb1, tpd, sl0, sl1, tbt,
                      isem0, isem1, osem0, osem1):
        wid = lax.axis_index("s") * NC + lax.axis_index("c")
        tb = (tb0, tb1)
        sl = (sl0, sl1)
        isem = (isem0, isem1)
        osem = (osem0, osem1)
        iota = lax.broadcasted_iota(jnp.int32, (16,), 0)
        iota_p = iota * pitch
        iota_ph = iota_p + 16 * pitch

        def col0(k):
            s = jnp.minimum(wid * per_w + k, last_s)
            return pl.multiple_of(s * TCOLS, 128)

        def start_a(k, p):
            pltpu.async_copy(tt_hbm.at[:, pl.ds(col0(k), TCOLS)], tb[p], isem[p])

        def wait_a(p):
            pltpu.make_async_copy(tt_hbm.at[:, pl.ds(0, TCOLS)], tb[p],
                                  isem[p]).wait()

        def start_c(k, p):
            pltpu.async_copy(sl[p],
                             out_hbm.at[pl.ds(pl.multiple_of(col0(k) * dim,
                                                             1024),
                                              TCOLS * dim)], osem[p])

        def wait_c(p):
            pltpu.make_async_copy(sl[p], out_hbm.at[pl.ds(0, TCOLS * dim)],
                                  osem[p]).wait()

        def repitch(tbuf, ncols):
            for cb in range(0, ncols, 128):
                span = min(128, ncols - cb)

                @plsc.parallel_loop(0, span, 16, unroll=8)
                def _(c1):
                    for e in range(dim):
                        tpd[pl.ds(e * pitch + cb + c1, 16)] = (
                            tbuf[e, pl.ds(cb + c1, 16)])

        def transpose(slab, ncols):
            @plsc.parallel_loop(0, ncols, unroll=8)
            def _(c):
                cvec = jnp.full((16,), c, jnp.int32)
                v_lo = plsc.load_gather(tpd, [iota_p + cvec])
                v_hi = plsc.load_gather(tpd, [iota_ph + cvec])
                slab[pl.ds(c * dim, 16)] = v_lo
                slab[pl.ds(c * dim + 16, 16)] = v_hi

        def work(p):
            repitch(tb[p], TCOLS)
            transpose(sl[p], TCOLS)

        start_a(0, 0)
        start_a(1, 1)
        wait_a(0)
        work(0)
        start_a(2, 0)
        start_c(0, 0)
        wait_a(1)
        work(1)
        start_a(3, 1)
        start_c(1, 1)

        @pl.loop(2, per_w - 2, step=2)
        def _(k0):
            for k, p in ((k0, 0), (k0 + 1, 1)):
                wait_a(p)
                wait_c(p)
                work(p)
                start_a(k + 2, p)
                start_c(k, p)

        for k, p in ((per_w - 2, 0), (per_w - 1, 1)):
            wait_a(p)
            wait_c(p)
            work(p)
            start_c(k, p)
        wait_c(0)
        wait_c(1)

        if rem:
            @pl.when(wid == 0)
            def _tail():
                c0t = vocab - rem
                pltpu.async_copy(tt_hbm.at[:, pl.ds(c0t, rem)], tbt, isem[0])
                pltpu.make_async_copy(tt_hbm.at[:, pl.ds(c0t, rem)], tbt,
                                      isem[0]).wait()
                repitch(tbt, rem)
                transpose(sl[0], rem)
                pltpu.async_copy(sl[0].at[pl.ds(0, rem * dim)],
                                 out_hbm.at[pl.ds(c0t * dim, rem * dim)],
                                 osem[0])
                pltpu.make_async_copy(sl[0].at[pl.ds(0, rem * dim)],
                                      out_hbm.at[pl.ds(c0t * dim, rem * dim)],
                                      osem[0]).wait()

    return repack_kernel


@functools.lru_cache(maxsize=None)
def _make_gather(n_l: int, n_b: int, dim: int, vocab: int):
    n_units = n_l * n_b // CH
    units_per_l = n_b // CH
    assert n_units % NW == 0
    per_w = n_units // NW
    assert per_w >= 6

    mesh = plsc.VectorSubcoreMesh(core_axis_name="c", subcore_axis_name="s")

    @functools.partial(
        pl.kernel,
        out_type=jax.ShapeDtypeStruct((n_l * (dim // 8), n_b // 128, 8, 128),
                                      jnp.float32),
        mesh=mesh,
        scratch_types=[
            pltpu.VMEM((CH,), jnp.int32),
            pltpu.VMEM((CH,), jnp.int32),
            pltpu.VMEM((CH, dim), jnp.float32),
            pltpu.VMEM((CH, dim), jnp.float32),
            pltpu.VMEM((dim // 8, CH // 128, 10, 129), jnp.float32),
            pltpu.VMEM((dim // 8, CH // 128, 10, 129), jnp.float32),
            pltpu.SemaphoreType.DMA,
            pltpu.SemaphoreType.DMA,
            pltpu.SemaphoreType.DMA,
            pltpu.SemaphoreType.DMA,
            pltpu.SemaphoreType.DMA,
            pltpu.SemaphoreType.DMA,
        ],
        compiler_params=pltpu.CompilerParams(
            use_tc_tiling_on_sc=False, needs_layout_passes=False,
            disable_bounds_checks=True
        ),
    )
    def gather_kernel(xf_hbm, table_hbm, out_hbm,
                      idx0, idx1, gb0, gb1, sl0, sl1,
                      isem0, isem1, gsem0, gsem1, osem0, osem1):
        wid = lax.axis_index("s") * NC + lax.axis_index("c")
        idx = (idx0, idx1)
        gb = (gb0, gb1)
        sl = (sl0, sl1)
        isem = (isem0, isem1)
        gsem = (gsem0, gsem1)
        osem = (osem0, osem1)
        iota = lax.broadcasted_iota(jnp.int32, (16,), 0)

        def start_a(k, p):
            off = (wid * per_w + k) * CH
            pltpu.async_copy(xf_hbm.at[pl.ds(off, CH)], idx[p], isem[p])

        def wait_a(p):
            pltpu.make_async_copy(xf_hbm.at[pl.ds(0, CH)], idx[p], isem[p]).wait()

        def start_b(p):
            pltpu.async_copy(table_hbm.at[idx[p]], gb[p], gsem[p])

        def wait_b(p):
            pltpu.make_async_copy(table_hbm.at[idx[p]], gb[p], gsem[p]).wait()

        nbh = CH // 128

        def out_slice(k):
            u = wid * per_w + k
            l = u // units_per_l
            bh0 = (u % units_per_l) * nbh
            return out_hbm.at[pl.ds(l * (dim // 8), dim // 8),
                              pl.ds(bh0, nbh), :, :]

        def slab_src(p):
            return sl[p].at[:, :, pl.ds(0, 8), pl.ds(0, 128)]

        def start_c(k, p):
            pltpu.async_copy(slab_src(p), out_slice(k), osem[p])

        def wait_c(p):
            pltpu.make_async_copy(slab_src(p), out_slice(0), osem[p]).wait()

        eh_lo = iota // 8
        eh_hi = eh_lo + 2
        el = iota % 8

        def transpose(p):
            gbuf = gb[p]
            slab = sl[p]

            @plsc.parallel_loop(0, CH, unroll=16)
            def _(r):
                v0 = gbuf[r, 0:16]
                v1 = gbuf[r, 16:32]
                bh = jnp.full((16,), r // 128, jnp.int32)
                bl = jnp.full((16,), r % 128, jnp.int32)
                plsc.store_scatter(slab, [eh_lo, bh, el, bl], v0)
                plsc.store_scatter(slab, [eh_hi, bh, el, bl], v1)

        start_a(0, 0)
        wait_a(0)
        start_b(0)
        start_a(1, 1)
        wait_a(1)
        start_b(1)
        wait_b(0)
        start_a(2, 0)
        transpose(0)
        start_c(0, 0)
        wait_a(0)
        start_b(0)
        wait_b(1)
        start_a(3, 1)
        transpose(1)
        start_c(1, 1)

        def steady(k, p):
            wait_a(p)
            start_b(p)
            wait_b(1 - p)
            start_a(k + 1, 1 - p)
            wait_c(1 - p)
            transpose(1 - p)
            start_c(k - 1, 1 - p)

        @pl.loop(3, per_w - 2, step=2)
        def _(k0):
            steady(k0, 1)
            steady(k0 + 1, 0)

        kl = per_w - 1
        wait_a(1)
        start_b(1)
        wait_b(0)
        wait_c(0)
        transpose(0)
        start_c(kl - 1, 0)
        wait_b(1)
        wait_c(1)
        transpose(1)
        start_c(kl, 1)
        wait_c(0)
        wait_c(1)

    return gather_kernel


def kernel(x, table):
    b, l = x.shape
    vocab, dim = table.shape
    flat_idx = x.T.reshape(l * b)
    table_lin = _make_repack(vocab, dim)(table.T).reshape(vocab, dim)
    out4 = _make_gather(l, b, dim, vocab)(flat_idx, table_lin)
    out5 = out4.reshape(l, dim // 8, b // 128, 8, 128)
    return out5.transpose(2, 4, 0, 1, 3).reshape(b, l, dim)


# speedup vs baseline: 1.2214x; 1.2214x over previous
import functools

import jax
import jax.numpy as jnp
from jax import lax
from jax.experimental import pallas as pl
from jax.experimental.pallas import tpu as pltpu
from jax.experimental.pallas import tpu_sc as plsc

NC = 2
NS = 16
NW = NC * NS

CH = 512
G = CH // 16

TCOLS = 512


@functools.lru_cache(maxsize=None)
def _make_repack(vocab: int, dim: int):
    n_full = vocab // TCOLS
    rem = vocab - n_full * TCOLS
    per_w = (n_full + NW - 1) // NW
    if per_w % 2:
        per_w += 1
    last_s = n_full - 1
    pitch = TCOLS + 1

    mesh = plsc.VectorSubcoreMesh(core_axis_name="c", subcore_axis_name="s")

    @functools.partial(
        pl.kernel,
        out_type=jax.ShapeDtypeStruct((vocab * dim,), jnp.float32),
        mesh=mesh,
        scratch_types=[
            pltpu.VMEM((dim, TCOLS), jnp.float32),
            pltpu.VMEM((dim, TCOLS), jnp.float32),
            pltpu.VMEM((dim * pitch,), jnp.float32),
            pltpu.VMEM((TCOLS * dim,), jnp.float32),
            pltpu.VMEM((TCOLS * dim,), jnp.float32),
            pltpu.VMEM((dim, max(rem, 16)), jnp.float32),
            pltpu.SemaphoreType.DMA,
            pltpu.SemaphoreType.DMA,
            pltpu.SemaphoreType.DMA,
            pltpu.SemaphoreType.DMA,
        ],
        compiler_params=pltpu.CompilerParams(
            use_tc_tiling_on_sc=True, needs_layout_passes=False,
            disable_bounds_checks=True
        ),
    )
    def repack_kernel(tt_hbm, out_hbm, tb0, tb1, tpd, sl0, sl1, tbt,
                      isem0, isem1, osem0, osem1):
        wid = lax.axis_index("s") * NC + lax.axis_index("c")
        tb = (tb0, tb1)
        sl = (sl0, sl1)
        isem = (isem0, isem1)
        osem = (osem0, osem1)
        iota = lax.broadcasted_iota(jnp.int32, (16,), 0)
        iota_p = iota * pitch
        iota_ph = iota_p + 16 * pitch

        def col0(k):
            s = jnp.minimum(wid * per_w + k, last_s)
            return pl.multiple_of(s * TCOLS, 128)

        def start_a(k, p):
            pltpu.async_copy(tt_hbm.at[:, pl.ds(col0(k), TCOLS)], tb[p], isem[p])

        def wait_a(p):
            pltpu.make_async_copy(tt_hbm.at[:, pl.ds(0, TCOLS)], tb[p],
                                  isem[p]).wait()

        def start_c(k, p):
            pltpu.async_copy(sl[p],
                             out_hbm.at[pl.ds(pl.multiple_of(col0(k) * dim,
                                                             1024),
                                              TCOLS * dim)], osem[p])

        def wait_c(p):
            pltpu.make_async_copy(sl[p], out_hbm.at[pl.ds(0, TCOLS * dim)],
                                  osem[p]).wait()

        def repitch(tbuf, ncols):
            for cb in range(0, ncols, 128):
                span = min(128, ncols - cb)

                @plsc.parallel_loop(0, span, 16, unroll=4)
                def _(c1):
                    for e in range(dim):
                        tpd[pl.ds(e * pitch + cb + c1, 16)] = (
                            tbuf[e, pl.ds(cb + c1, 16)])

        def transpose(slab, ncols):
            @plsc.parallel_loop(0, ncols, unroll=8)
            def _(c):
                cvec = jnp.full((16,), c, jnp.int32)
                v_lo = plsc.load_gather(tpd, [iota_p + cvec])
                v_hi = plsc.load_gather(tpd, [iota_ph + cvec])
                slab[pl.ds(c * dim, 16)] = v_lo
                slab[pl.ds(c * dim + 16, 16)] = v_hi

        def work(p):
            repitch(tb[p], TCOLS)
            transpose(sl[p], TCOLS)

        start_a(0, 0)
        start_a(1, 1)
        wait_a(0)
        work(0)
        start_a(2, 0)
        start_c(0, 0)
        wait_a(1)
        work(1)
        start_a(3, 1)
        start_c(1, 1)

        @pl.loop(2, per_w - 2, step=2)
        def _(k0):
            for k, p in ((k0, 0), (k0 + 1, 1)):
                wait_a(p)
                wait_c(p)
                work(p)
                start_a(k + 2, p)
                start_c(k, p)

        for k, p in ((per_w - 2, 0), (per_w - 1, 1)):
            wait_a(p)
            wait_c(p)
            work(p)
            start_c(k, p)
        wait_c(0)
        wait_c(1)

        if rem:
            @pl.when(wid == 0)
            def _tail():
                c0t = vocab - rem
                pltpu.async_copy(tt_hbm.at[:, pl.ds(c0t, rem)], tbt, isem[0])
                pltpu.make_async_copy(tt_hbm.at[:, pl.ds(c0t, rem)], tbt,
                                      isem[0]).wait()
                repitch(tbt, rem)
                transpose(sl[0], rem)
                pltpu.async_copy(sl[0].at[pl.ds(0, rem * dim)],
                                 out_hbm.at[pl.ds(c0t * dim, rem * dim)],
                                 osem[0])
                pltpu.make_async_copy(sl[0].at[pl.ds(0, rem * dim)],
                                      out_hbm.at[pl.ds(c0t * dim, rem * dim)],
                                      osem[0]).wait()

    return repack_kernel


@functools.lru_cache(maxsize=None)
def _make_gather(n_l: int, n_b: int, dim: int, vocab: int):
    n_units = n_l * n_b // CH
    units_per_l = n_b // CH
    assert n_units % NW == 0
    per_w = n_units // NW
    assert per_w >= 6

    mesh = plsc.VectorSubcoreMesh(core_axis_name="c", subcore_axis_name="s")

    @functools.partial(
        pl.kernel,
        out_type=jax.ShapeDtypeStruct((n_l * (dim // 8), n_b // 128, 8, 128),
                                      jnp.float32),
        mesh=mesh,
        scratch_types=[
            pltpu.VMEM((CH,), jnp.int32),
            pltpu.VMEM((CH,), jnp.int32),
            pltpu.VMEM((CH, dim), jnp.float32),
            pltpu.VMEM((CH, dim), jnp.float32),
            pltpu.VMEM((dim // 8, CH // 128, 10, 129), jnp.float32),
            pltpu.VMEM((dim // 8, CH // 128, 10, 129), jnp.float32),
            pltpu.SemaphoreType.DMA,
            pltpu.SemaphoreType.DMA,
            pltpu.SemaphoreType.DMA,
            pltpu.SemaphoreType.DMA,
            pltpu.SemaphoreType.DMA,
            pltpu.SemaphoreType.DMA,
        ],
        compiler_params=pltpu.CompilerParams(
            use_tc_tiling_on_sc=False, needs_layout_passes=False,
            disable_bounds_checks=True
        ),
    )
    def gather_kernel(xf_hbm, table_hbm, out_hbm,
                      idx0, idx1, gb0, gb1, sl0, sl1,
                      isem0, isem1, gsem0, gsem1, osem0, osem1):
        wid = lax.axis_index("s") * NC + lax.axis_index("c")
        idx = (idx0, idx1)
        gb = (gb0, gb1)
        sl = (sl0, sl1)
        isem = (isem0, isem1)
        gsem = (gsem0, gsem1)
        osem = (osem0, osem1)
        iota = lax.broadcasted_iota(jnp.int32, (16,), 0)

        def start_a(k, p):
            off = (wid * per_w + k) * CH
            pltpu.async_copy(xf_hbm.at[pl.ds(off, CH)], idx[p], isem[p])

        def wait_a(p):
            pltpu.make_async_copy(xf_hbm.at[pl.ds(0, CH)], idx[p], isem[p]).wait()

        def start_b(p):
            pltpu.async_copy(table_hbm.at[idx[p]], gb[p], gsem[p])

        def wait_b(p):
            pltpu.make_async_copy(table_hbm.at[idx[p]], gb[p], gsem[p]).wait()

        nbh = CH // 128

        def out_slice(k):
            u = wid * per_w + k
            l = u // units_per_l
            bh0 = (u % units_per_l) * nbh
            return out_hbm.at[pl.ds(l * (dim // 8), dim // 8),
                              pl.ds(bh0, nbh), :, :]

        def slab_src(p):
            return sl[p].at[:, :, pl.ds(0, 8), pl.ds(0, 128)]

        def start_c(k, p):
            pltpu.async_copy(slab_src(p), out_slice(k), osem[p])

        def wait_c(p):
            pltpu.make_async_copy(slab_src(p), out_slice(0), osem[p]).wait()

        eh_lo = iota // 8
        eh_hi = eh_lo + 2
        el = iota % 8

        def transpose(p):
            gbuf = gb[p]
            slab = sl[p]

            @plsc.parallel_loop(0, CH, unroll=8)
            def _(r):
                v0 = gbuf[r, 0:16]
                v1 = gbuf[r, 16:32]
                bh = jnp.full((16,), r // 128, jnp.int32)
                bl = jnp.full((16,), r % 128, jnp.int32)
                plsc.store_scatter(slab, [eh_lo, bh, el, bl], v0)
                plsc.store_scatter(slab, [eh_hi, bh, el, bl], v1)

        start_a(0, 0)
        wait_a(0)
        start_b(0)
        start_a(1, 1)
        wait_a(1)
        start_b(1)
        wait_b(0)
        start_a(2, 0)
        transpose(0)
        start_c(0, 0)
        wait_a(0)
        start_b(0)
        wait_b(1)
        start_a(3, 1)
        transpose(1)
        start_c(1, 1)

        def steady(k, p):
            wait_a(p)
            start_b(p)
            wait_b(1 - p)
            start_a(k + 1, 1 - p)
            wait_c(1 - p)
            transpose(1 - p)
            start_c(k - 1, 1 - p)

        @pl.loop(3, per_w - 2, step=2)
        def _(k0):
            steady(k0, 1)
            steady(k0 + 1, 0)

        kl = per_w - 1
        wait_a(1)
        start_b(1)
        wait_b(0)
        wait_c(0)
        transpose(0)
        start_c(kl - 1, 0)
        wait_b(1)
        wait_c(1)
        transpose(1)
        start_c(kl, 1)
        wait_c(0)
        wait_c(1)

    return gather_kernel


def kernel(x, table):
    b, l = x.shape
    vocab, dim = table.shape
    flat_idx = x.T.reshape(l * b)
    table_lin = _make_repack(vocab, dim)(table.T).reshape(vocab, dim)
    out4 = _make_gather(l, b, dim, vocab)(flat_idx, table_lin)
    out5 = out4.reshape(l, dim // 8, b // 128, 8, 128)
    return out5.transpose(2, 4, 0, 1, 3).reshape(b, l, dim)
